# Initial kernel scaffold; baseline (speedup 1.0000x reference)
import jax
import jax.numpy as jnp
from jax.experimental import pallas as pl


def kernel(x, edge_index, W1, b1, W2, b2):
    raise NotImplementedError("write your pallas kernel here")


# speedup vs baseline: 11.8441x; 11.8441x over previous
import functools

import jax
import jax.numpy as jnp
from jax import lax
from jax.experimental import pallas as pl
from jax.experimental.pallas import tpu as pltpu
from jax.experimental.pallas import tpu_sc as plsc

N = 10000
E = 320000
D = 128
NC = 2
NS = 16
NW = NC * NS
CHUNK = 125
ECH = E // CHUNK
CPW = ECH // NW
NPAD = 10240
RPT = NPAD // NS
DEGW = 128


def _s_op(u, srcc, dstc, zrows):
    mesh = plsc.VectorSubcoreMesh(core_axis_name="c", subcore_axis_name="s")

    @functools.partial(
        pl.kernel,
        out_type=jax.ShapeDtypeStruct((NC, NPAD, D), jnp.float32),
        mesh=mesh,
        scratch_types=[
            pltpu.VMEM((CPW, CHUNK), jnp.int32),
            pltpu.VMEM((CPW, CHUNK), jnp.int32),
            pltpu.VMEM((CHUNK, D), jnp.float32),
            pltpu.VMEM_SHARED((NPAD, D), jnp.float32),
            pltpu.SemaphoreType.DMA,
        ],
    )
    def k(u_hbm, src_hbm, dst_hbm, z_hbm, out_hbm, srcb, dstb, rows, acc, sem):
        cid = lax.axis_index("c")
        sid = lax.axis_index("s")
        wid = sid * NC + cid
        r0 = pl.multiple_of(sid * RPT, 8)
        pltpu.sync_copy(z_hbm, acc.at[pl.ds(r0, RPT)])
        c0 = pl.multiple_of(wid * CPW, 8)
        pltpu.sync_copy(src_hbm.at[pl.ds(c0, CPW)], srcb)
        pltpu.sync_copy(dst_hbm.at[pl.ds(c0, CPW)], dstb)
        plsc.subcore_barrier()

        def body(i, carry):
            pltpu.async_copy(u_hbm.at[srcb.at[i]], rows, sem).wait()
            pltpu.sync_copy(rows, acc.at[dstb.at[i]], add=True)
            return carry

        lax.fori_loop(0, CPW, body, 0)
        plsc.subcore_barrier()
        pltpu.sync_copy(acc.at[pl.ds(r0, RPT)], out_hbm.at[cid, pl.ds(r0, RPT)])

    return k(u, srcc, dstc, zrows)


def _deg_op(srcc, onesd, zdeg):
    mesh = plsc.VectorSubcoreMesh(core_axis_name="c", subcore_axis_name="s")

    @functools.partial(
        pl.kernel,
        out_type=jax.ShapeDtypeStruct((NC, NPAD, DEGW), jnp.float32),
        mesh=mesh,
        scratch_types=[
            pltpu.VMEM((CPW, CHUNK), jnp.int32),
            pltpu.VMEM((CHUNK, DEGW), jnp.float32),
            pltpu.VMEM_SHARED((NPAD, DEGW), jnp.float32),
        ],
    )
    def k(src_hbm, ones_hbm, z_hbm, out_hbm, srcb, onesb, acc):
        cid = lax.axis_index("c")
        sid = lax.axis_index("s")
        wid = sid * NC + cid
        r0 = pl.multiple_of(sid * RPT, 8)
        pltpu.sync_copy(z_hbm, acc.at[pl.ds(r0, RPT)])
        c0 = pl.multiple_of(wid * CPW, 8)
        pltpu.sync_copy(src_hbm.at[pl.ds(c0, CPW)], srcb)
        pltpu.sync_copy(ones_hbm, onesb)
        plsc.subcore_barrier()

        def body(i, carry):
            pltpu.sync_copy(onesb, acc.at[srcb.at[i]], add=True)
            return carry

        lax.fori_loop(0, CPW, body, 0)
        plsc.subcore_barrier()
        pltpu.sync_copy(acc.at[pl.ds(r0, RPT)], out_hbm.at[cid, pl.ds(r0, RPT)])

    return k(srcc, onesd, zdeg)


BN = 1000


def _tc_pre(degp, x):
    def body(degp_ref, x_ref, dinv_ref, u0_ref):
        deg = degp_ref[0][:, 0:1] + degp_ref[1][:, 0:1]
        dinv = jnp.where(deg > 0, lax.rsqrt(deg), 0.0)
        dinv_ref[...] = jnp.broadcast_to(dinv, (BN, D))
        u0_ref[...] = x_ref[...] * dinv

    return pl.pallas_call(
        body,
        grid=(N // BN,),
        in_specs=[
            pl.BlockSpec((NC, BN, DEGW), lambda i: (0, i, 0)),
            pl.BlockSpec((BN, D), lambda i: (i, 0)),
        ],
        out_specs=[pl.BlockSpec((BN, D), lambda i: (i, 0))] * 2,
        out_shape=[jax.ShapeDtypeStruct((N, D), jnp.float32)] * 2,
    )(degp, x)


def _tc_mid1(P, dinvb, t0, W):
    def body(p_ref, dinv_ref, t0_ref, w_ref, u_ref, acc_ref):
        dv = dinv_ref[...]
        tx1 = -(dv * (p_ref[0] + p_ref[1]))
        u_ref[...] = dv * tx1
        acc_ref[...] = jnp.dot(
            t0_ref[...], w_ref[0], preferred_element_type=jnp.float32
        ) + jnp.dot(tx1, w_ref[1], preferred_element_type=jnp.float32)

    return pl.pallas_call(
        body,
        grid=(N // BN,),
        in_specs=[
            pl.BlockSpec((NC, BN, D), lambda i: (0, i, 0)),
            pl.BlockSpec((BN, D), lambda i: (i, 0)),
            pl.BlockSpec((BN, D), lambda i: (i, 0)),
            pl.BlockSpec((3, D, D), lambda i: (0, 0, 0)),
        ],
        out_specs=[pl.BlockSpec((BN, D), lambda i: (i, 0))] * 2,
        out_shape=[jax.ShapeDtypeStruct((N, D), jnp.float32)] * 2,
    )(P, dinvb, t0, W)


def _tc_mid2(P, dinvb, t0, acc, W, b):
    def body(p_ref, dinv_ref, t0_ref, acc_ref, w_ref, b_ref, h_ref, u_ref):
        dv = dinv_ref[...]
        tx2 = -2.0 * dv * (p_ref[0] + p_ref[1]) - t0_ref[...]
        pre = (
            acc_ref[...]
            + jnp.dot(tx2, w_ref[2], preferred_element_type=jnp.float32)
            + b_ref[...]
        )
        h = jnp.maximum(pre, 0.0)
        h_ref[...] = h
        u_ref[...] = dv * h

    return pl.pallas_call(
        body,
        grid=(N // BN,),
        in_specs=[
            pl.BlockSpec((NC, BN, D), lambda i: (0, i, 0)),
            pl.BlockSpec((BN, D), lambda i: (i, 0)),
            pl.BlockSpec((BN, D), lambda i: (i, 0)),
            pl.BlockSpec((BN, D), lambda i: (i, 0)),
            pl.BlockSpec((3, D, D), lambda i: (0, 0, 0)),
            pl.BlockSpec((1, D), lambda i: (0, 0)),
        ],
        out_specs=[pl.BlockSpec((BN, D), lambda i: (i, 0))] * 2,
        out_shape=[jax.ShapeDtypeStruct((N, D), jnp.float32)] * 2,
    )(P, dinvb, t0, acc, W, b)


def _tc_fin(P, dinvb, t0, acc, W, b):
    def body(p_ref, dinv_ref, t0_ref, acc_ref, w_ref, b_ref, o_ref):
        dv = dinv_ref[...]
        ty2 = -2.0 * dv * (p_ref[0] + p_ref[1]) - t0_ref[...]
        o_ref[...] = (
            acc_ref[...]
            + jnp.dot(ty2, w_ref[2], preferred_element_type=jnp.float32)
            + b_ref[...]
        )

    return pl.pallas_call(
        body,
        grid=(N // BN,),
        in_specs=[
            pl.BlockSpec((NC, BN, D), lambda i: (0, i, 0)),
            pl.BlockSpec((BN, D), lambda i: (i, 0)),
            pl.BlockSpec((BN, D), lambda i: (i, 0)),
            pl.BlockSpec((BN, D), lambda i: (i, 0)),
            pl.BlockSpec((3, D, D), lambda i: (0, 0, 0)),
            pl.BlockSpec((1, D), lambda i: (0, 0)),
        ],
        out_specs=pl.BlockSpec((BN, D), lambda i: (i, 0)),
        out_shape=jax.ShapeDtypeStruct((N, D), jnp.float32),
    )(P, dinvb, t0, acc, W, b)


def kernel(x, edge_index, W1, b1, W2, b2):
    src = edge_index[0]
    dst = edge_index[1]
    srcc = src.reshape(ECH, CHUNK)
    dstc = dst.reshape(ECH, CHUNK)
    zrows = jnp.zeros((RPT, D), jnp.float32)
    zdeg = jnp.zeros((RPT, DEGW), jnp.float32)
    onesd = jnp.ones((CHUNK, DEGW), jnp.float32)
    b1r = b1.reshape(1, D)
    b2r = b2.reshape(1, D)

    degp = _deg_op(srcc, onesd, zdeg)
    dinvb, u0 = _tc_pre(degp, x)
    P1 = _s_op(u0, srcc, dstc, zrows)
    u1, acc1 = _tc_mid1(P1, dinvb, x, W1)
    P2 = _s_op(u1, srcc, dstc, zrows)
    h, u2 = _tc_mid2(P2, dinvb, x, acc1, W1, b1r)
    P3 = _s_op(u2, srcc, dstc, zrows)
    u3, acc2 = _tc_mid1(P3, dinvb, h, W2)
    P4 = _s_op(u3, srcc, dstc, zrows)
    return _tc_fin(P4, dinvb, h, acc2, W2, b2r)


# speedup vs baseline: 15.0149x; 1.2677x over previous
import functools

import jax
import jax.numpy as jnp
from jax import lax
from jax.experimental import pallas as pl
from jax.experimental.pallas import tpu as pltpu
from jax.experimental.pallas import tpu_sc as plsc

N = 10000
E = 320000
D = 128
NC = 2
NS = 16
NW = NC * NS
CHUNK = 125
ECH = E // CHUNK
CPW = ECH // NW
NPAD = 10240
RPT = NPAD // NS
DEGW = 128


def _s_op(u, eidx, zrows):
    mesh = plsc.VectorSubcoreMesh(core_axis_name="c", subcore_axis_name="s")

    @functools.partial(
        pl.kernel,
        out_type=jax.ShapeDtypeStruct((NC, NPAD, D), jnp.float32),
        mesh=mesh,
        scratch_types=[
            [pltpu.VMEM((2, CHUNK), jnp.int32) for _ in range(4)],
            [pltpu.VMEM((CHUNK, D), jnp.float32) for _ in range(2)],
            pltpu.VMEM_SHARED((NPAD, D), jnp.float32),
            [pltpu.SemaphoreType.DMA for _ in range(4)],
            [pltpu.SemaphoreType.DMA for _ in range(2)],
        ],
    )
    def k(u_hbm, ei_hbm, z_hbm, out_hbm, idxb, rows, acc, isem, rsem):
        cid = lax.axis_index("c")
        sid = lax.axis_index("s")
        wid = sid * NC + cid
        r0 = pl.multiple_of(sid * RPT, 8)
        pltpu.sync_copy(z_hbm, acc.at[pl.ds(r0, RPT)])
        c0 = wid * CPW
        cmax = c0 + CPW - 1

        def idx_start(c, b):
            pltpu.async_copy(ei_hbm.at[c], idxb[b], isem[b])

        def idx_wait(b):
            pltpu.make_async_copy(ei_hbm.at[c0], idxb[b], isem[b]).wait()

        def gather_start(b, r):
            pltpu.async_copy(u_hbm.at[idxb[b].at[0]], rows[r], rsem[r])

        def gather_wait(r):
            pltpu.make_async_copy(u_hbm.at[idxb[0].at[0]], rows[r], rsem[r]).wait()

        def scatter(b, r):
            pltpu.sync_copy(rows[r], acc.at[idxb[b].at[1]], add=True)

        plsc.subcore_barrier()
        for b in range(4):
            idx_start(c0 + b, b)

        def body(j, carry):
            base = c0 + 4 * j
            idx_wait(0)
            gather_start(0, 0)
            idx_wait(1)
            gather_start(1, 1)
            gather_wait(0)
            scatter(0, 0)
            idx_start(jnp.minimum(base + 4, cmax), 0)
            idx_wait(2)
            gather_start(2, 0)
            gather_wait(1)
            scatter(1, 1)
            idx_start(jnp.minimum(base + 5, cmax), 1)
            idx_wait(3)
            gather_start(3, 1)
            gather_wait(0)
            scatter(2, 0)
            idx_start(jnp.minimum(base + 6, cmax), 2)
            gather_wait(1)
            scatter(3, 1)
            idx_start(jnp.minimum(base + 7, cmax), 3)
            return carry

        lax.fori_loop(0, CPW // 4, body, 0)
        for b in range(4):
            idx_wait(b)
        plsc.subcore_barrier()
        pltpu.sync_copy(acc.at[pl.ds(r0, RPT)], out_hbm.at[cid, pl.ds(r0, RPT)])

    return k(u, eidx, zrows)


def _deg_op(srcc, onesd, zdeg):
    mesh = plsc.VectorSubcoreMesh(core_axis_name="c", subcore_axis_name="s")

    @functools.partial(
        pl.kernel,
        out_type=jax.ShapeDtypeStruct((NC, NPAD, DEGW), jnp.float32),
        mesh=mesh,
        scratch_types=[
            pltpu.VMEM((CPW, CHUNK), jnp.int32),
            pltpu.VMEM((CHUNK, DEGW), jnp.float32),
            pltpu.VMEM_SHARED((NPAD, DEGW), jnp.float32),
        ],
    )
    def k(src_hbm, ones_hbm, z_hbm, out_hbm, srcb, onesb, acc):
        cid = lax.axis_index("c")
        sid = lax.axis_index("s")
        wid = sid * NC + cid
        r0 = pl.multiple_of(sid * RPT, 8)
        pltpu.sync_copy(z_hbm, acc.at[pl.ds(r0, RPT)])
        c0 = pl.multiple_of(wid * CPW, 8)
        pltpu.sync_copy(src_hbm.at[pl.ds(c0, CPW)], srcb)
        pltpu.sync_copy(ones_hbm, onesb)
        plsc.subcore_barrier()

        def body(i, carry):
            pltpu.sync_copy(onesb, acc.at[srcb.at[i]], add=True)
            return carry

        lax.fori_loop(0, CPW, body, 0)
        plsc.subcore_barrier()
        pltpu.sync_copy(acc.at[pl.ds(r0, RPT)], out_hbm.at[cid, pl.ds(r0, RPT)])

    return k(srcc, onesd, zdeg)


BN = 1000


def _tc_pre(degp, x):
    def body(degp_ref, x_ref, dinv_ref, u0_ref):
        deg = degp_ref[0][:, 0:1] + degp_ref[1][:, 0:1]
        dinv = jnp.where(deg > 0, lax.rsqrt(deg), 0.0)
        dinv_ref[...] = jnp.broadcast_to(dinv, (BN, D))
        u0_ref[...] = x_ref[...] * dinv

    return pl.pallas_call(
        body,
        grid=(N // BN,),
        in_specs=[
            pl.BlockSpec((NC, BN, DEGW), lambda i: (0, i, 0)),
            pl.BlockSpec((BN, D), lambda i: (i, 0)),
        ],
        out_specs=[pl.BlockSpec((BN, D), lambda i: (i, 0))] * 2,
        out_shape=[jax.ShapeDtypeStruct((N, D), jnp.float32)] * 2,
    )(degp, x)


def _tc_mid1(P, dinvb, t0, W):
    def body(p_ref, dinv_ref, t0_ref, w_ref, u_ref, acc_ref):
        dv = dinv_ref[...]
        tx1 = -(dv * (p_ref[0] + p_ref[1]))
        u_ref[...] = dv * tx1
        acc_ref[...] = jnp.dot(
            t0_ref[...], w_ref[0], preferred_element_type=jnp.float32
        ) + jnp.dot(tx1, w_ref[1], preferred_element_type=jnp.float32)

    return pl.pallas_call(
        body,
        grid=(N // BN,),
        in_specs=[
            pl.BlockSpec((NC, BN, D), lambda i: (0, i, 0)),
            pl.BlockSpec((BN, D), lambda i: (i, 0)),
            pl.BlockSpec((BN, D), lambda i: (i, 0)),
            pl.BlockSpec((3, D, D), lambda i: (0, 0, 0)),
        ],
        out_specs=[pl.BlockSpec((BN, D), lambda i: (i, 0))] * 2,
        out_shape=[jax.ShapeDtypeStruct((N, D), jnp.float32)] * 2,
    )(P, dinvb, t0, W)


def _tc_mid2(P, dinvb, t0, acc, W, b):
    def body(p_ref, dinv_ref, t0_ref, acc_ref, w_ref, b_ref, h_ref, u_ref):
        dv = dinv_ref[...]
        tx2 = -2.0 * dv * (p_ref[0] + p_ref[1]) - t0_ref[...]
        pre = (
            acc_ref[...]
            + jnp.dot(tx2, w_ref[2], preferred_element_type=jnp.float32)
            + b_ref[...]
        )
        h = jnp.maximum(pre, 0.0)
        h_ref[...] = h
        u_ref[...] = dv * h

    return pl.pallas_call(
        body,
        grid=(N // BN,),
        in_specs=[
            pl.BlockSpec((NC, BN, D), lambda i: (0, i, 0)),
            pl.BlockSpec((BN, D), lambda i: (i, 0)),
            pl.BlockSpec((BN, D), lambda i: (i, 0)),
            pl.BlockSpec((BN, D), lambda i: (i, 0)),
            pl.BlockSpec((3, D, D), lambda i: (0, 0, 0)),
            pl.BlockSpec((1, D), lambda i: (0, 0)),
        ],
        out_specs=[pl.BlockSpec((BN, D), lambda i: (i, 0))] * 2,
        out_shape=[jax.ShapeDtypeStruct((N, D), jnp.float32)] * 2,
    )(P, dinvb, t0, acc, W, b)


def _tc_fin(P, dinvb, t0, acc, W, b):
    def body(p_ref, dinv_ref, t0_ref, acc_ref, w_ref, b_ref, o_ref):
        dv = dinv_ref[...]
        ty2 = -2.0 * dv * (p_ref[0] + p_ref[1]) - t0_ref[...]
        o_ref[...] = (
            acc_ref[...]
            + jnp.dot(ty2, w_ref[2], preferred_element_type=jnp.float32)
            + b_ref[...]
        )

    return pl.pallas_call(
        body,
        grid=(N // BN,),
        in_specs=[
            pl.BlockSpec((NC, BN, D), lambda i: (0, i, 0)),
            pl.BlockSpec((BN, D), lambda i: (i, 0)),
            pl.BlockSpec((BN, D), lambda i: (i, 0)),
            pl.BlockSpec((BN, D), lambda i: (i, 0)),
            pl.BlockSpec((3, D, D), lambda i: (0, 0, 0)),
            pl.BlockSpec((1, D), lambda i: (0, 0)),
        ],
        out_specs=pl.BlockSpec((BN, D), lambda i: (i, 0)),
        out_shape=jax.ShapeDtypeStruct((N, D), jnp.float32),
    )(P, dinvb, t0, acc, W, b)


def kernel(x, edge_index, W1, b1, W2, b2):
    src = edge_index[0]
    srcc = src.reshape(ECH, CHUNK)
    eidx = edge_index.reshape(2, ECH, CHUNK).transpose(1, 0, 2)
    zrows = jnp.zeros((RPT, D), jnp.float32)
    zdeg = jnp.zeros((RPT, DEGW), jnp.float32)
    onesd = jnp.ones((CHUNK, DEGW), jnp.float32)
    b1r = b1.reshape(1, D)
    b2r = b2.reshape(1, D)

    degp = _deg_op(srcc, onesd, zdeg)
    dinvb, u0 = _tc_pre(degp, x)
    P1 = _s_op(u0, eidx, zrows)
    u1, acc1 = _tc_mid1(P1, dinvb, x, W1)
    P2 = _s_op(u1, eidx, zrows)
    h, u2 = _tc_mid2(P2, dinvb, x, acc1, W1, b1r)
    P3 = _s_op(u2, eidx, zrows)
    u3, acc2 = _tc_mid1(P3, dinvb, h, W2)
    P4 = _s_op(u3, eidx, zrows)
    return _tc_fin(P4, dinvb, h, acc2, W2, b2r)


# speedup vs baseline: 15.3015x; 1.0191x over previous
import functools

import jax
import jax.numpy as jnp
from jax import lax
from jax.experimental import pallas as pl
from jax.experimental.pallas import tpu as pltpu
from jax.experimental.pallas import tpu_sc as plsc

N = 10000
E = 320000
D = 128
NC = 2
NS = 16
NW = NC * NS
CHUNK = 125
ECH = E // CHUNK
CPW = ECH // NW
NPAD = 10240
RPT = NPAD // NS
DEGW = 128


def _s_op(u, eidx, zrows):
    mesh = plsc.VectorSubcoreMesh(core_axis_name="c", subcore_axis_name="s")

    @functools.partial(
        pl.kernel,
        out_type=jax.ShapeDtypeStruct((NC, NPAD, D), jnp.float32),
        mesh=mesh,
        scratch_types=[
            [pltpu.VMEM((2, CHUNK), jnp.int32) for _ in range(4)],
            [pltpu.VMEM((CHUNK, D), jnp.float32) for _ in range(2)],
            pltpu.VMEM_SHARED((NPAD, D), jnp.float32),
            [pltpu.SemaphoreType.DMA for _ in range(4)],
            [pltpu.SemaphoreType.DMA for _ in range(2)],
        ],
    )
    def k(u_hbm, ei_hbm, z_hbm, out_hbm, idxb, rows, acc, isem, rsem):
        cid = lax.axis_index("c")
        sid = lax.axis_index("s")
        wid = sid * NC + cid
        r0 = pl.multiple_of(sid * RPT, 8)
        c0 = wid * CPW
        cmax = c0 + CPW - 1

        def idx_start(c, b):
            pltpu.async_copy(ei_hbm.at[c], idxb[b], isem[b])

        def idx_wait(b):
            pltpu.make_async_copy(ei_hbm.at[c0], idxb[b], isem[b]).wait()

        def gather_start(b, r):
            pltpu.async_copy(u_hbm.at[idxb[b].at[0]], rows[r], rsem[r])

        def gather_wait(r):
            pltpu.make_async_copy(u_hbm.at[idxb[0].at[0]], rows[r], rsem[r]).wait()

        def scatter(b, r):
            pltpu.sync_copy(rows[r], acc.at[idxb[b].at[1]], add=True)

        for b in range(4):
            idx_start(c0 + b, b)
        pltpu.sync_copy(z_hbm, acc.at[pl.ds(r0, RPT)])
        plsc.subcore_barrier()

        def body(j, carry):
            base = c0 + 4 * j
            idx_wait(0)
            gather_start(0, 0)
            idx_wait(1)
            gather_start(1, 1)
            gather_wait(0)
            scatter(0, 0)
            idx_start(jnp.minimum(base + 4, cmax), 0)
            idx_wait(2)
            gather_start(2, 0)
            gather_wait(1)
            scatter(1, 1)
            idx_start(jnp.minimum(base + 5, cmax), 1)
            idx_wait(3)
            gather_start(3, 1)
            gather_wait(0)
            scatter(2, 0)
            idx_start(jnp.minimum(base + 6, cmax), 2)
            gather_wait(1)
            scatter(3, 1)
            idx_start(jnp.minimum(base + 7, cmax), 3)
            return carry

        lax.fori_loop(0, CPW // 4, body, 0)
        for b in range(4):
            idx_wait(b)
        plsc.subcore_barrier()
        pltpu.sync_copy(acc.at[pl.ds(r0, RPT)], out_hbm.at[cid, pl.ds(r0, RPT)])

    return k(u, eidx, zrows)


def _deg_op(srcc, onesd, zdeg):
    mesh = plsc.VectorSubcoreMesh(core_axis_name="c", subcore_axis_name="s")

    @functools.partial(
        pl.kernel,
        out_type=jax.ShapeDtypeStruct((NC, NPAD, DEGW), jnp.float32),
        mesh=mesh,
        scratch_types=[
            pltpu.VMEM((CPW, CHUNK), jnp.int32),
            pltpu.VMEM((CHUNK, DEGW), jnp.float32),
            pltpu.VMEM_SHARED((NPAD, DEGW), jnp.float32),
        ],
    )
    def k(src_hbm, ones_hbm, z_hbm, out_hbm, srcb, onesb, acc):
        cid = lax.axis_index("c")
        sid = lax.axis_index("s")
        wid = sid * NC + cid
        r0 = pl.multiple_of(sid * RPT, 8)
        pltpu.sync_copy(z_hbm, acc.at[pl.ds(r0, RPT)])
        c0 = pl.multiple_of(wid * CPW, 8)
        pltpu.sync_copy(src_hbm.at[pl.ds(c0, CPW)], srcb)
        pltpu.sync_copy(ones_hbm, onesb)
        plsc.subcore_barrier()

        def body(i, carry):
            pltpu.sync_copy(onesb, acc.at[srcb.at[i]], add=True)
            return carry

        lax.fori_loop(0, CPW, body, 0)
        plsc.subcore_barrier()
        pltpu.sync_copy(acc.at[pl.ds(r0, RPT)], out_hbm.at[cid, pl.ds(r0, RPT)])

    return k(srcc, onesd, zdeg)


BN = 2000


def _tc_pre(degp, x):
    def body(degp_ref, x_ref, dinv_ref, u0_ref):
        deg = degp_ref[0][:, 0:1] + degp_ref[1][:, 0:1]
        dinv = jnp.where(deg > 0, lax.rsqrt(deg), 0.0)
        dinv_ref[...] = jnp.broadcast_to(dinv, (BN, D))
        u0_ref[...] = x_ref[...] * dinv

    return pl.pallas_call(
        body,
        grid=(N // BN,),
        in_specs=[
            pl.BlockSpec((NC, BN, DEGW), lambda i: (0, i, 0)),
            pl.BlockSpec((BN, D), lambda i: (i, 0)),
        ],
        out_specs=[pl.BlockSpec((BN, D), lambda i: (i, 0))] * 2,
        out_shape=[jax.ShapeDtypeStruct((N, D), jnp.float32)] * 2,
    )(degp, x)


def _tc_mid1(P, dinvb, t0, W):
    def body(p_ref, dinv_ref, t0_ref, w_ref, u_ref, acc_ref):
        dv = dinv_ref[...]
        tx1 = -(dv * (p_ref[0] + p_ref[1]))
        u_ref[...] = dv * tx1
        acc_ref[...] = jnp.dot(
            t0_ref[...], w_ref[0], preferred_element_type=jnp.float32
        ) + jnp.dot(tx1, w_ref[1], preferred_element_type=jnp.float32)

    return pl.pallas_call(
        body,
        grid=(N // BN,),
        in_specs=[
            pl.BlockSpec((NC, BN, D), lambda i: (0, i, 0)),
            pl.BlockSpec((BN, D), lambda i: (i, 0)),
            pl.BlockSpec((BN, D), lambda i: (i, 0)),
            pl.BlockSpec((3, D, D), lambda i: (0, 0, 0)),
        ],
        out_specs=[pl.BlockSpec((BN, D), lambda i: (i, 0))] * 2,
        out_shape=[jax.ShapeDtypeStruct((N, D), jnp.float32)] * 2,
    )(P, dinvb, t0, W)


def _tc_mid2(P, dinvb, t0, acc, W, b):
    def body(p_ref, dinv_ref, t0_ref, acc_ref, w_ref, b_ref, h_ref, u_ref):
        dv = dinv_ref[...]
        tx2 = -2.0 * dv * (p_ref[0] + p_ref[1]) - t0_ref[...]
        pre = (
            acc_ref[...]
            + jnp.dot(tx2, w_ref[2], preferred_element_type=jnp.float32)
            + b_ref[...]
        )
        h = jnp.maximum(pre, 0.0)
        h_ref[...] = h
        u_ref[...] = dv * h

    return pl.pallas_call(
        body,
        grid=(N // BN,),
        in_specs=[
            pl.BlockSpec((NC, BN, D), lambda i: (0, i, 0)),
            pl.BlockSpec((BN, D), lambda i: (i, 0)),
            pl.BlockSpec((BN, D), lambda i: (i, 0)),
            pl.BlockSpec((BN, D), lambda i: (i, 0)),
            pl.BlockSpec((3, D, D), lambda i: (0, 0, 0)),
            pl.BlockSpec((1, D), lambda i: (0, 0)),
        ],
        out_specs=[pl.BlockSpec((BN, D), lambda i: (i, 0))] * 2,
        out_shape=[jax.ShapeDtypeStruct((N, D), jnp.float32)] * 2,
    )(P, dinvb, t0, acc, W, b)


def _tc_fin(P, dinvb, t0, acc, W, b):
    def body(p_ref, dinv_ref, t0_ref, acc_ref, w_ref, b_ref, o_ref):
        dv = dinv_ref[...]
        ty2 = -2.0 * dv * (p_ref[0] + p_ref[1]) - t0_ref[...]
        o_ref[...] = (
            acc_ref[...]
            + jnp.dot(ty2, w_ref[2], preferred_element_type=jnp.float32)
            + b_ref[...]
        )

    return pl.pallas_call(
        body,
        grid=(N // BN,),
        in_specs=[
            pl.BlockSpec((NC, BN, D), lambda i: (0, i, 0)),
            pl.BlockSpec((BN, D), lambda i: (i, 0)),
            pl.BlockSpec((BN, D), lambda i: (i, 0)),
            pl.BlockSpec((BN, D), lambda i: (i, 0)),
            pl.BlockSpec((3, D, D), lambda i: (0, 0, 0)),
            pl.BlockSpec((1, D), lambda i: (0, 0)),
        ],
        out_specs=pl.BlockSpec((BN, D), lambda i: (i, 0)),
        out_shape=jax.ShapeDtypeStruct((N, D), jnp.float32),
    )(P, dinvb, t0, acc, W, b)


def kernel(x, edge_index, W1, b1, W2, b2):
    src = edge_index[0]
    srcc = src.reshape(ECH, CHUNK)
    eidx = edge_index.reshape(2, ECH, CHUNK).transpose(1, 0, 2)
    zrows = jnp.zeros((RPT, D), jnp.float32)
    zdeg = jnp.zeros((RPT, DEGW), jnp.float32)
    onesd = jnp.ones((CHUNK, DEGW), jnp.float32)
    b1r = b1.reshape(1, D)
    b2r = b2.reshape(1, D)

    degp = _deg_op(srcc, onesd, zdeg)
    dinvb, u0 = _tc_pre(degp, x)
    P1 = _s_op(u0, eidx, zrows)
    u1, acc1 = _tc_mid1(P1, dinvb, x, W1)
    P2 = _s_op(u1, eidx, zrows)
    h, u2 = _tc_mid2(P2, dinvb, x, acc1, W1, b1r)
    P3 = _s_op(u2, eidx, zrows)
    u3, acc2 = _tc_mid1(P3, dinvb, h, W2)
    P4 = _s_op(u3, eidx, zrows)
    return _tc_fin(P4, dinvb, h, acc2, W2, b2r)


# speedup vs baseline: 17.4853x; 1.1427x over previous
import functools

import jax
import jax.numpy as jnp
from jax import lax
from jax.experimental import pallas as pl
from jax.experimental.pallas import tpu as pltpu
from jax.experimental.pallas import tpu_sc as plsc

N = 10000
E = 320000
D = 128
NC = 2
NS = 16
NW = NC * NS
CHUNK = 125
ECH = E // CHUNK
CPW = ECH // NW
NPAD = 10240
RPT = NPAD // NS
DEGW = 128


def _s_op(u, eidx, zrows):
    mesh = plsc.VectorSubcoreMesh(core_axis_name="c", subcore_axis_name="s")

    @functools.partial(
        pl.kernel,
        out_type=jax.ShapeDtypeStruct((NC, NPAD, D), jnp.float32),
        mesh=mesh,
        scratch_types=[
            [pltpu.VMEM((4, 2, CHUNK), jnp.int32) for _ in range(2)],
            [pltpu.VMEM((CHUNK, D), jnp.float32) for _ in range(2)],
            pltpu.VMEM_SHARED((NPAD, D), jnp.float32),
            [pltpu.SemaphoreType.DMA for _ in range(2)],
            [pltpu.SemaphoreType.DMA for _ in range(2)],
        ],
    )
    def k(u_hbm, ei_hbm, z_hbm, out_hbm, ibuf, rows, acc, isem, rsem):
        cid = lax.axis_index("c")
        sid = lax.axis_index("s")
        wid = sid * NC + cid
        r0 = pl.multiple_of(sid * RPT, 8)
        c0 = wid * CPW
        cq = c0 + CPW - 4

        def ifetch_start(cb, p):
            pltpu.async_copy(ei_hbm.at[pl.ds(cb, 4)], ibuf[p], isem[p])

        def ifetch_wait(p):
            pltpu.make_async_copy(ei_hbm.at[pl.ds(c0, 4)], ibuf[p], isem[p]).wait()

        def gather_start(p, t, r):
            pltpu.async_copy(u_hbm.at[ibuf[p].at[t, 0]], rows[r], rsem[r])

        def gather_wait(r):
            pltpu.make_async_copy(u_hbm.at[ibuf[0].at[0, 0]], rows[r], rsem[r]).wait()

        def scatter(p, t, r):
            pltpu.sync_copy(rows[r], acc.at[ibuf[p].at[t, 1]], add=True)

        ifetch_start(c0, 0)
        pltpu.sync_copy(z_hbm, acc.at[pl.ds(r0, RPT)])
        plsc.subcore_barrier()
        ifetch_wait(0)
        gather_start(0, 0, 0)
        ifetch_start(c0 + 4, 1)

        def body(j, carry):
            base = c0 + 8 * j
            gather_start(0, 1, 1)
            gather_wait(0)
            scatter(0, 0, 0)
            gather_start(0, 2, 0)
            gather_wait(1)
            scatter(0, 1, 1)
            gather_start(0, 3, 1)
            gather_wait(0)
            scatter(0, 2, 0)
            ifetch_wait(1)
            gather_start(1, 0, 0)
            gather_wait(1)
            scatter(0, 3, 1)
            ifetch_start(jnp.minimum(base + 8, cq), 0)
            gather_start(1, 1, 1)
            gather_wait(0)
            scatter(1, 0, 0)
            gather_start(1, 2, 0)
            gather_wait(1)
            scatter(1, 1, 1)
            gather_start(1, 3, 1)
            gather_wait(0)
            scatter(1, 2, 0)
            ifetch_wait(0)
            gather_start(0, 0, 0)
            gather_wait(1)
            scatter(1, 3, 1)
            ifetch_start(jnp.minimum(base + 12, cq), 1)
            return carry

        lax.fori_loop(0, CPW // 8, body, 0)
        gather_wait(0)
        ifetch_wait(1)
        plsc.subcore_barrier()
        pltpu.sync_copy(acc.at[pl.ds(r0, RPT)], out_hbm.at[cid, pl.ds(r0, RPT)])

    return k(u, eidx, zrows)


def _deg_op(eidx, onesd, zdeg):
    mesh = plsc.VectorSubcoreMesh(core_axis_name="c", subcore_axis_name="s")

    @functools.partial(
        pl.kernel,
        out_type=jax.ShapeDtypeStruct((NC, NPAD, DEGW), jnp.float32),
        mesh=mesh,
        scratch_types=[
            [pltpu.VMEM((8, 2, CHUNK), jnp.int32) for _ in range(2)],
            pltpu.VMEM((CHUNK, DEGW), jnp.float32),
            pltpu.VMEM_SHARED((NPAD, DEGW), jnp.float32),
            [pltpu.SemaphoreType.DMA for _ in range(2)],
            pltpu.SemaphoreType.DMA,
        ],
    )
    def k(ei_hbm, ones_hbm, z_hbm, out_hbm, ibuf, onesb, acc, isem, ssem):
        cid = lax.axis_index("c")
        sid = lax.axis_index("s")
        wid = sid * NC + cid
        r0 = pl.multiple_of(sid * RPT, 8)
        c0 = wid * CPW

        def ifetch_start(cb, p):
            pltpu.async_copy(ei_hbm.at[pl.ds(cb, 8)], ibuf[p], isem[p])

        def ifetch_wait(p):
            pltpu.make_async_copy(ei_hbm.at[pl.ds(c0, 8)], ibuf[p], isem[p]).wait()

        ifetch_start(c0, 0)
        pltpu.sync_copy(ones_hbm, onesb)
        pltpu.sync_copy(z_hbm, acc.at[pl.ds(r0, RPT)])
        plsc.subcore_barrier()

        def group(p):
            for t in range(8):
                pltpu.async_copy(onesb, acc.at[ibuf[p].at[t, 0]], ssem, add=True)
            for t in range(8):
                pltpu.make_async_copy(onesb, acc.at[ibuf[p].at[t, 0]], ssem).wait()

        def body(j, carry):
            base = c0 + 16 * j
            ifetch_start(base + 8, 1)
            ifetch_wait(0)
            group(0)
            ifetch_start(jnp.minimum(base + 16, c0 + CPW - 8), 0)
            ifetch_wait(1)
            group(1)
            return carry

        lax.fori_loop(0, CPW // 16, body, 0)
        ifetch_wait(0)
        plsc.subcore_barrier()
        pltpu.sync_copy(acc.at[pl.ds(r0, RPT)], out_hbm.at[cid, pl.ds(r0, RPT)])

    return k(eidx, onesd, zdeg)


BN = 2000


def _tc_pre(degp, x):
    def body(degp_ref, x_ref, dinv_ref, u0_ref):
        deg = degp_ref[0][:, 0:1] + degp_ref[1][:, 0:1]
        dinv = jnp.where(deg > 0, lax.rsqrt(deg), 0.0)
        dinv_ref[...] = jnp.broadcast_to(dinv, (BN, D))
        u0_ref[...] = x_ref[...] * dinv

    return pl.pallas_call(
        body,
        grid=(N // BN,),
        in_specs=[
            pl.BlockSpec((NC, BN, DEGW), lambda i: (0, i, 0)),
            pl.BlockSpec((BN, D), lambda i: (i, 0)),
        ],
        out_specs=[pl.BlockSpec((BN, D), lambda i: (i, 0))] * 2,
        out_shape=[jax.ShapeDtypeStruct((N, D), jnp.float32)] * 2,
    )(degp, x)


def _tc_mid1(P, dinvb, t0, W):
    def body(p_ref, dinv_ref, t0_ref, w_ref, u_ref, acc_ref):
        dv = dinv_ref[...]
        tx1 = -(dv * (p_ref[0] + p_ref[1]))
        u_ref[...] = dv * tx1
        acc_ref[...] = jnp.dot(
            t0_ref[...], w_ref[0], preferred_element_type=jnp.float32
        ) + jnp.dot(tx1, w_ref[1], preferred_element_type=jnp.float32)

    return pl.pallas_call(
        body,
        grid=(N // BN,),
        in_specs=[
            pl.BlockSpec((NC, BN, D), lambda i: (0, i, 0)),
            pl.BlockSpec((BN, D), lambda i: (i, 0)),
            pl.BlockSpec((BN, D), lambda i: (i, 0)),
            pl.BlockSpec((3, D, D), lambda i: (0, 0, 0)),
        ],
        out_specs=[pl.BlockSpec((BN, D), lambda i: (i, 0))] * 2,
        out_shape=[jax.ShapeDtypeStruct((N, D), jnp.float32)] * 2,
    )(P, dinvb, t0, W)


def _tc_mid2(P, dinvb, t0, acc, W, b):
    def body(p_ref, dinv_ref, t0_ref, acc_ref, w_ref, b_ref, h_ref, u_ref):
        dv = dinv_ref[...]
        tx2 = -2.0 * dv * (p_ref[0] + p_ref[1]) - t0_ref[...]
        pre = (
            acc_ref[...]
            + jnp.dot(tx2, w_ref[2], preferred_element_type=jnp.float32)
            + b_ref[...]
        )
        h = jnp.maximum(pre, 0.0)
        h_ref[...] = h
        u_ref[...] = dv * h

    return pl.pallas_call(
        body,
        grid=(N // BN,),
        in_specs=[
            pl.BlockSpec((NC, BN, D), lambda i: (0, i, 0)),
            pl.BlockSpec((BN, D), lambda i: (i, 0)),
            pl.BlockSpec((BN, D), lambda i: (i, 0)),
            pl.BlockSpec((BN, D), lambda i: (i, 0)),
            pl.BlockSpec((3, D, D), lambda i: (0, 0, 0)),
            pl.BlockSpec((1, D), lambda i: (0, 0)),
        ],
        out_specs=[pl.BlockSpec((BN, D), lambda i: (i, 0))] * 2,
        out_shape=[jax.ShapeDtypeStruct((N, D), jnp.float32)] * 2,
    )(P, dinvb, t0, acc, W, b)


def _tc_fin(P, dinvb, t0, acc, W, b):
    def body(p_ref, dinv_ref, t0_ref, acc_ref, w_ref, b_ref, o_ref):
        dv = dinv_ref[...]
        ty2 = -2.0 * dv * (p_ref[0] + p_ref[1]) - t0_ref[...]
        o_ref[...] = (
            acc_ref[...]
            + jnp.dot(ty2, w_ref[2], preferred_element_type=jnp.float32)
            + b_ref[...]
        )

    return pl.pallas_call(
        body,
        grid=(N // BN,),
        in_specs=[
            pl.BlockSpec((NC, BN, D), lambda i: (0, i, 0)),
            pl.BlockSpec((BN, D), lambda i: (i, 0)),
            pl.BlockSpec((BN, D), lambda i: (i, 0)),
            pl.BlockSpec((BN, D), lambda i: (i, 0)),
            pl.BlockSpec((3, D, D), lambda i: (0, 0, 0)),
            pl.BlockSpec((1, D), lambda i: (0, 0)),
        ],
        out_specs=pl.BlockSpec((BN, D), lambda i: (i, 0)),
        out_shape=jax.ShapeDtypeStruct((N, D), jnp.float32),
    )(P, dinvb, t0, acc, W, b)


def kernel(x, edge_index, W1, b1, W2, b2):
    eidx = edge_index.reshape(2, ECH, CHUNK).transpose(1, 0, 2)
    zrows = jnp.zeros((RPT, D), jnp.float32)
    zdeg = jnp.zeros((RPT, DEGW), jnp.float32)
    onesd = jnp.ones((CHUNK, DEGW), jnp.float32)
    b1r = b1.reshape(1, D)
    b2r = b2.reshape(1, D)

    degp = _deg_op(eidx, onesd, zdeg)
    dinvb, u0 = _tc_pre(degp, x)
    P1 = _s_op(u0, eidx, zrows)
    u1, acc1 = _tc_mid1(P1, dinvb, x, W1)
    P2 = _s_op(u1, eidx, zrows)
    h, u2 = _tc_mid2(P2, dinvb, x, acc1, W1, b1r)
    P3 = _s_op(u2, eidx, zrows)
    u3, acc2 = _tc_mid1(P3, dinvb, h, W2)
    P4 = _s_op(u3, eidx, zrows)
    return _tc_fin(P4, dinvb, h, acc2, W2, b2r)
